# baseline (speedup 1.0000x reference)
import functools

import jax
import jax.numpy as jnp
from jax import lax
from jax.experimental import pallas as pl
from jax.experimental.pallas import tpu as pltpu
from jax.experimental.pallas import tpu_sc as plsc

B = 16384
D = 32
L = 16
NC = 2
NS = 16
NW = NC * NS
BPW = B // NW
CHUNK = 128
NCHUNK = BPW // CHUNK


def _gmf_body(uidx_hbm, iidx_hbm, w_hbm, b_hbm, utab_hbm, itab_hbm, out_hbm,
              uidx_v, iidx_v, urows_v, irows_v, w_v, b_v, out_v, sem_u, sem_i):
    wid = lax.axis_index("s") * NC + lax.axis_index("c")
    base = wid * BPW

    pltpu.sync_copy(uidx_hbm.at[pl.ds(wid * NCHUNK, NCHUNK)], uidx_v)
    pltpu.sync_copy(iidx_hbm.at[pl.ds(wid * NCHUNK, NCHUNK)], iidx_v)

    copies = []
    for c in range(NCHUNK):
        copies.append(pltpu.async_copy(utab_hbm.at[uidx_v.at[c]], urows_v.at[c], sem_u))
        copies.append(pltpu.async_copy(itab_hbm.at[iidx_v.at[c]], irows_v.at[c], sem_i))
    pltpu.sync_copy(w_hbm, w_v)
    pltpu.sync_copy(b_hbm, b_v)
    for cp in copies:
        cp.wait()

    bvec = b_v[...]
    iota = lax.iota(jnp.int32, L)
    wvecs = [w_v[d] for d in range(D)]

    def body(g, carry):
        c = g // (CHUNK // L)
        rows = (g % (CHUNK // L)) * L + iota
        cvec = jnp.full((L,), 0, jnp.int32) + c
        acc = bvec
        for d in range(D):
            dvec = jnp.full((L,), d, jnp.int32)
            uu = plsc.load_gather(urows_v, [cvec, rows, dvec])
            ii = plsc.load_gather(irows_v, [cvec, rows, dvec])
            acc = acc + uu * ii * wvecs[d]
        out_v[pl.ds(g * L, L)] = acc
        return carry

    lax.fori_loop(0, BPW // L, body, 0)
    pltpu.sync_copy(out_v, out_hbm.at[pl.ds(base, BPW)])


def _gmf_call(uidx2d, iidx2d, w_flat, b16, user_table, item_table):
    mesh = plsc.VectorSubcoreMesh(core_axis_name="c", subcore_axis_name="s")
    kern = functools.partial(
        pl.kernel,
        mesh=mesh,
        compiler_params=pltpu.CompilerParams(
            needs_layout_passes=False, use_tc_tiling_on_sc=False),
        out_type=jax.ShapeDtypeStruct((B,), jnp.float32),
        scratch_types=[
            pltpu.VMEM((NCHUNK, CHUNK), jnp.int32),
            pltpu.VMEM((NCHUNK, CHUNK), jnp.int32),
            pltpu.VMEM((NCHUNK, CHUNK, D), jnp.float32),
            pltpu.VMEM((NCHUNK, CHUNK, D), jnp.float32),
            pltpu.VMEM((D, L), jnp.float32),
            pltpu.VMEM((L,), jnp.float32),
            pltpu.VMEM((BPW,), jnp.float32),
            pltpu.SemaphoreType.DMA,
            pltpu.SemaphoreType.DMA,
        ],
    )(_gmf_body)
    return kern(uidx2d, iidx2d, w_flat, b16, user_table, item_table)


def kernel(user_indices, item_indices, ratings, user_table, item_table, W, b):
    del ratings
    uidx2d = user_indices.reshape(NW * NCHUNK, CHUNK)
    iidx2d = item_indices.reshape(NW * NCHUNK, CHUNK)
    wb = jnp.broadcast_to(W, (D, L))
    b16 = jnp.broadcast_to(b, (L,))
    return _gmf_call(uidx2d, iidx2d, wb, b16, user_table, item_table)
